# Initial kernel scaffold; baseline (speedup 1.0000x reference)
import jax
import jax.numpy as jnp
from jax.experimental import pallas as pl


def kernel(memory, last_update_t, edge_feats, edge_timestamps, time_w, time_b, W_ih, W_hh, b_ih, b_hh, edge_index):
    raise NotImplementedError("write your pallas kernel here")


# speedup vs baseline: 21.0063x; 21.0063x over previous
import functools

import jax
import jax.numpy as jnp
from jax import lax
from jax.experimental import pallas as pl
from jax.experimental.pallas import tpu as pltpu
from jax.experimental.pallas import tpu_sc as plsc

N = 10000
E = 320000
MEM_DIM = 128
E_FEAT = 16
T_DIM = 100

NC = 2
NS = 16
NW = NC * NS
NPAD = 10240
EPW = E // NW
NPW = NPAD // NW
LANES = 16
GCHUNK = 80

_MESH = plsc.VectorSubcoreMesh(core_axis_name="c", subcore_axis_name="s")
_SC_PARAMS = pltpu.CompilerParams(needs_layout_passes=False,
                                  use_tc_tiling_on_sc=False)


def _wid():
    return lax.axis_index("s") * NC + lax.axis_index("c")


def _fill(ref, n, value):
    vec = jnp.full((LANES,), value, dtype=ref.dtype)

    def body(j, _):
        ref[pl.ds(j * LANES, LANES)] = vec
        return 0

    lax.fori_loop(0, n // LANES, body, 0)


_GDN = lax.GatherDimensionNumbers(
    offset_dims=(), collapsed_slice_dims=(0,), start_index_map=(0,))


def _take16(x, idx):
    return lax.gather(x, idx[:, None], dimension_numbers=_GDN,
                      slice_sizes=(1,),
                      mode=lax.GatherScatterMode.PROMISE_IN_BOUNDS)


def _scatter_max(loc_ref, idx, val, active, neg):
    lane = lax.iota(jnp.int32, LANES)
    v = jnp.where(active, val, neg)
    sk, sv = plsc.sort_key_val(idx, v)
    for k in (1, 2, 4, 8):
        rot = (lane + k) & (LANES - 1)
        k_k = _take16(sk, rot)
        v_k = _take16(sv, rot)
        sv = jnp.maximum(sv, jnp.where(sk == k_k, v_k, neg))
    prev = _take16(sk, (lane - 1) & (LANES - 1))
    head = (lane == 0) | (sk != prev)
    cur = plsc.load_gather(loc_ref, [sk])
    new = jnp.maximum(cur, sv)
    plsc.store_scatter(loc_ref, [sk], new, mask=head)


@functools.partial(
    pl.kernel,
    out_type=jax.ShapeDtypeStruct((NW, NPAD), jnp.float32),
    mesh=_MESH,
    compiler_params=_SC_PARAMS,
    scratch_types=[
        pltpu.VMEM((EPW,), jnp.int32),
        pltpu.VMEM((EPW,), jnp.float32),
        pltpu.VMEM((NPAD,), jnp.float32),
    ],
)
def _k_seg_max_ts(dst_hbm, ts_hbm, part_hbm, dst_v, ts_v, loc_v):
    wid = _wid()
    base = wid * EPW
    pltpu.sync_copy(dst_hbm.at[pl.ds(base, EPW)], dst_v)
    pltpu.sync_copy(ts_hbm.at[pl.ds(base, EPW)], ts_v)
    _fill(loc_v, NPAD, -jnp.inf)

    def body(i, _):
        dstv = dst_v[pl.ds(i * LANES, LANES)]
        tsv = ts_v[pl.ds(i * LANES, LANES)]
        _scatter_max(loc_v, dstv, tsv, jnp.ones((LANES,), dtype=jnp.bool_),
                     -jnp.inf)
        return 0

    lax.fori_loop(0, EPW // LANES, body, 0)
    pltpu.sync_copy(loc_v, part_hbm.at[wid])


@functools.partial(
    pl.kernel,
    out_type=jax.ShapeDtypeStruct((NPAD,), jnp.float32),
    mesh=_MESH,
    compiler_params=_SC_PARAMS,
    scratch_types=[
        pltpu.VMEM((NW, NPW), jnp.float32),
        pltpu.VMEM((NPW,), jnp.float32),
    ],
)
def _k_merge_max(part_hbm, maxts_hbm, blk_v, out_v):
    wid = _wid()
    base = wid * NPW
    pltpu.sync_copy(part_hbm.at[:, pl.ds(base, NPW)], blk_v)

    def body(j, _):
        acc = blk_v[0, pl.ds(j * LANES, LANES)]
        for r in range(1, NW):
            acc = jnp.maximum(acc, blk_v[r, pl.ds(j * LANES, LANES)])
        out_v[pl.ds(j * LANES, LANES)] = acc
        return 0

    lax.fori_loop(0, NPW // LANES, body, 0)
    pltpu.sync_copy(out_v, maxts_hbm.at[pl.ds(base, NPW)])


@functools.partial(
    pl.kernel,
    out_type=jax.ShapeDtypeStruct((NW, NPAD), jnp.int32),
    mesh=_MESH,
    compiler_params=_SC_PARAMS,
    scratch_types=[
        pltpu.VMEM((EPW,), jnp.int32),
        pltpu.VMEM((EPW,), jnp.float32),
        pltpu.VMEM((NPAD,), jnp.float32),
        pltpu.VMEM((NPAD,), jnp.int32),
    ],
)
def _k_seg_argmax(dst_hbm, ts_hbm, maxts_hbm, part_hbm, dst_v, ts_v, maxts_v, loc_v):
    wid = _wid()
    base = wid * EPW
    pltpu.sync_copy(dst_hbm.at[pl.ds(base, EPW)], dst_v)
    pltpu.sync_copy(ts_hbm.at[pl.ds(base, EPW)], ts_v)
    pltpu.sync_copy(maxts_hbm, maxts_v)
    _fill(loc_v, NPAD, -1)
    lane = lax.iota(jnp.int32, LANES)

    def body(i, _):
        dstv = dst_v[pl.ds(i * LANES, LANES)]
        tsv = ts_v[pl.ds(i * LANES, LANES)]
        mts = plsc.load_gather(maxts_v, [dstv])
        eidv = base + i * LANES + lane
        _scatter_max(loc_v, dstv, eidv, tsv >= mts, jnp.int32(-1))
        return 0

    lax.fori_loop(0, EPW // LANES, body, 0)
    pltpu.sync_copy(loc_v, part_hbm.at[wid])


@functools.partial(
    pl.kernel,
    out_type=(
        jax.ShapeDtypeStruct((NPAD,), jnp.int32),
        jax.ShapeDtypeStruct((NPAD,), jnp.float32),
        jax.ShapeDtypeStruct((NPAD,), jnp.float32),
        jax.ShapeDtypeStruct((NPAD, MEM_DIM), jnp.float32),
        jax.ShapeDtypeStruct((NPAD, E_FEAT), jnp.float32),
    ),
    mesh=_MESH,
    compiler_params=_SC_PARAMS,
    scratch_types=[
        pltpu.VMEM((NW, NPW), jnp.int32),
        pltpu.VMEM((NPW,), jnp.int32),
        pltpu.VMEM((NPW,), jnp.int32),
        pltpu.VMEM((NPW,), jnp.float32),
        pltpu.VMEM((NPW,), jnp.int32),
        pltpu.VMEM((NPW,), jnp.float32),
        pltpu.VMEM((NPW,), jnp.float32),
        pltpu.VMEM((NPW, MEM_DIM), jnp.float32),
        pltpu.VMEM((NPW, E_FEAT), jnp.float32),
        pltpu.SemaphoreType.DMA,
    ],
)
def _k_gather(part_hbm, src_hbm, ts_hbm, feats_hbm, mem_hbm, lu_hbm,
              best_hbm, tsb_hbm, delta_hbm, memsrc_hbm, featsb_hbm,
              blk_v, safe_v, best_v, tsb_v, srcs_v, lus_v, delta_v,
              memsrc_v, featsb_v, sem):
    wid = _wid()
    base = wid * NPW
    pltpu.sync_copy(part_hbm.at[:, pl.ds(base, NPW)], blk_v)

    def merge(j, _):
        sl = pl.ds(j * LANES, LANES)
        acc = blk_v[0, sl]
        for r in range(1, NW):
            acc = jnp.maximum(acc, blk_v[r, sl])
        best_v[sl] = acc
        safe_v[sl] = jnp.maximum(acc, 0)
        return 0

    lax.fori_loop(0, NPW // LANES, merge, 0)
    pltpu.sync_copy(best_v, best_hbm.at[pl.ds(base, NPW)])

    for c in range(0, NPW, GCHUNK):
        idx = safe_v.at[pl.ds(c, GCHUNK)]
        pltpu.async_copy(ts_hbm.at[idx], tsb_v.at[pl.ds(c, GCHUNK)], sem).wait()
        pltpu.async_copy(src_hbm.at[idx], srcs_v.at[pl.ds(c, GCHUNK)], sem).wait()
        pltpu.async_copy(feats_hbm.at[idx], featsb_v.at[pl.ds(c, GCHUNK)], sem).wait()
    for c in range(0, NPW, GCHUNK):
        idx = srcs_v.at[pl.ds(c, GCHUNK)]
        pltpu.async_copy(mem_hbm.at[idx], memsrc_v.at[pl.ds(c, GCHUNK)], sem).wait()
        pltpu.async_copy(lu_hbm.at[idx], lus_v.at[pl.ds(c, GCHUNK)], sem).wait()

    def dbody(j, _):
        sl = pl.ds(j * LANES, LANES)
        delta_v[sl] = tsb_v[sl] - lus_v[sl]
        return 0

    lax.fori_loop(0, NPW // LANES, dbody, 0)

    pltpu.sync_copy(tsb_v, tsb_hbm.at[pl.ds(base, NPW)])
    pltpu.sync_copy(delta_v, delta_hbm.at[pl.ds(base, NPW)])
    pltpu.sync_copy(memsrc_v, memsrc_hbm.at[pl.ds(base, NPW)])
    pltpu.sync_copy(featsb_v, featsb_hbm.at[pl.ds(base, NPW)])


BLK = 256
GRID = NPAD // BLK
H3 = 3 * MEM_DIM


def _gru_body(mem_ref, memsrc_ref, feats_ref, delta_ref, best_ref, tsb_ref,
              lu_ref, wvec_ref, tb_ref, wa_ref, wb_ref, wc_ref, wd_ref,
              whh_ref, bih_ref, bhh_ref, outm_ref, outt_ref):
    h = mem_ref[...]
    delta = delta_ref[...]
    te = jnp.cos(delta * wvec_ref[...] + tb_ref[...])
    dn = (((1,), (1,)), ((), ()))
    dot = functools.partial(lax.dot_general, dimension_numbers=dn,
                            preferred_element_type=jnp.float32,
                            precision=lax.Precision.HIGHEST)
    gi = (dot(memsrc_ref[...], wa_ref[...]) + dot(h, wb_ref[...])
          + dot(feats_ref[...], wc_ref[...]) + dot(te, wd_ref[...])
          + bih_ref[...])
    gh = dot(h, whh_ref[...]) + bhh_ref[...]
    r = jax.nn.sigmoid(gi[:, 0:MEM_DIM] + gh[:, 0:MEM_DIM])
    z = jax.nn.sigmoid(gi[:, MEM_DIM:2 * MEM_DIM] + gh[:, MEM_DIM:2 * MEM_DIM])
    n = jnp.tanh(gi[:, 2 * MEM_DIM:] + r * gh[:, 2 * MEM_DIM:])
    new = (1.0 - z) * n + z * h
    has = best_ref[...] >= 0
    outm_ref[...] = jnp.where(has, new, h)
    outt_ref[...] = jnp.where(has, tsb_ref[...], lu_ref[...])


def _row_spec(width):
    return pl.BlockSpec((BLK, width), lambda i: (i, 0))


def _full_spec(shape):
    return pl.BlockSpec(shape, lambda i: tuple(0 for _ in shape))


_gru_call = pl.pallas_call(
    _gru_body,
    grid=(GRID,),
    in_specs=[
        _row_spec(MEM_DIM),
        _row_spec(MEM_DIM),
        _row_spec(E_FEAT),
        _row_spec(1),
        _row_spec(1),
        _row_spec(1),
        _row_spec(1),
        _full_spec((1, MEM_DIM)),
        _full_spec((1, MEM_DIM)),
        _full_spec((H3, MEM_DIM)),
        _full_spec((H3, MEM_DIM)),
        _full_spec((H3, E_FEAT)),
        _full_spec((H3, MEM_DIM)),
        _full_spec((H3, MEM_DIM)),
        _full_spec((1, H3)),
        _full_spec((1, H3)),
    ],
    out_specs=[_row_spec(MEM_DIM), _row_spec(1)],
    out_shape=[
        jax.ShapeDtypeStruct((NPAD, MEM_DIM), jnp.float32),
        jax.ShapeDtypeStruct((NPAD, 1), jnp.float32),
    ],
    compiler_params=pltpu.CompilerParams(
        dimension_semantics=("arbitrary",),
    ),
)


def kernel(memory, last_update_t, edge_feats, edge_timestamps, time_w, time_b,
           W_ih, W_hh, b_ih, b_hh, edge_index):
    src = edge_index[0].astype(jnp.int32)
    dst = edge_index[1].astype(jnp.int32)
    ts = edge_timestamps.astype(jnp.float32)

    part_max = _k_seg_max_ts(dst, ts)
    maxts = _k_merge_max(part_max)
    part_best = _k_seg_argmax(dst, ts, maxts)
    best, tsb, delta, memsrc, featsb = _k_gather(
        part_best, src, ts, edge_feats, memory, last_update_t)

    wa = W_ih[:, 0:MEM_DIM]
    wb = W_ih[:, MEM_DIM:2 * MEM_DIM]
    wc = W_ih[:, 2 * MEM_DIM:2 * MEM_DIM + E_FEAT]
    wd = jnp.pad(W_ih[:, 2 * MEM_DIM + E_FEAT:], ((0, 0), (0, MEM_DIM - T_DIM)))
    wvec = jnp.pad(time_w[:, 0], (0, MEM_DIM - T_DIM))[None, :]
    tbvec = jnp.pad(time_b, (0, MEM_DIM - T_DIM))[None, :]
    mem_pad = jnp.pad(memory, ((0, NPAD - N), (0, 0)))
    lu_pad = jnp.pad(last_update_t, (0, NPAD - N))

    out_mem, out_ts = _gru_call(
        mem_pad, memsrc, featsb, delta[:, None], best[:, None], tsb[:, None],
        lu_pad[:, None], wvec, tbvec, wa, wb, wc, wd, W_hh,
        b_ih[None, :], b_hh[None, :])
    return out_mem[:N], out_ts[:N, 0]
